# Initial kernel scaffold; baseline (speedup 1.0000x reference)
import jax
import jax.numpy as jnp
from jax.experimental import pallas as pl


def kernel(x, w_gate, W1, b1, W2, b2, gamma):
    raise NotImplementedError("write your pallas kernel here")


# speedup vs baseline: 3.4244x; 3.4244x over previous
import functools

import jax
import jax.numpy as jnp
from jax.experimental import pallas as pl
from jax.experimental.pallas import tpu as pltpu

def _gelu_exact(v):
    return 0.5 * v * (1.0 + jax.lax.erf(v * (2.0 ** -0.5)))


B, P, D = 1, 2048, 768
E, K, H = 8, 2, 768
TN = 256
GRID = P // TN


def _moe_block(x_ref, wg_ref, w1_ref, b1_ref, w2_ref, b2_ref, gamma_ref,
               y_ref, loss_ref, acc_ref):
    t = pl.program_id(0)
    xb = x_ref[...]

    logits = jnp.dot(xb, wg_ref[...], preferred_element_type=jnp.float32)
    p = jax.nn.softmax(logits, axis=1)
    iota = jax.lax.broadcasted_iota(jnp.int32, (TN, E), 1)
    a1 = jnp.argmax(p, axis=1)
    oh1 = iota == a1[:, None]
    pm = jnp.where(oh1, -1.0, p)
    a2 = jnp.argmax(pm, axis=1)
    oh2 = iota == a2[:, None]
    m1 = jnp.max(p, axis=1, keepdims=True)
    m2 = jnp.max(pm, axis=1, keepdims=True)
    den = m1 + m2 + 1e-6
    g1 = m1 / den
    g2 = m2 / den
    wmat = jnp.where(oh1, g1, 0.0) + jnp.where(oh2, g2, 0.0)

    imp = jnp.sum(wmat, axis=0)
    load = jnp.sum((wmat > 0).astype(jnp.float32), axis=0)

    @pl.when(t == 0)
    def _():
        acc_ref[...] = jnp.zeros_like(acc_ref)

    acc_ref[0, :] += imp
    acc_ref[1, :] += load

    xb16 = xb.astype(jnp.bfloat16)
    acc = jnp.zeros((TN, D), dtype=jnp.float32)
    for e in range(E):
        h = jnp.dot(xb16, w1_ref[e], preferred_element_type=jnp.float32)
        h = _gelu_exact(h + b1_ref[e][None, :])
        o = jnp.dot(h.astype(jnp.bfloat16), w2_ref[e],
                    preferred_element_type=jnp.float32)
        o = o + b2_ref[e][None, :]
        acc = acc + wmat[:, e][:, None] * o

    y = xb + acc
    norm = jnp.sqrt(jnp.sum(y * y, axis=1, keepdims=True))
    y = y / jnp.maximum(norm, 1e-12) * gamma_ref[...] * (float(D) ** 0.5)
    y_ref[...] = _gelu_exact(y)

    @pl.when(t == GRID - 1)
    def _():
        def cv_sq(v):
            mean = jnp.sum(v) / E
            var = jnp.sum((v - mean) ** 2) / (E - 1)
            return var / (mean * mean + 1e-10)

        loss_ref[...] = (cv_sq(acc_ref[0, :]) + cv_sq(acc_ref[1, :])).reshape(1, 1)


@jax.jit
def kernel(x, w_gate, W1, b1, W2, b2, gamma):
    x_flat = x.reshape(P, D)
    w1b = W1.astype(jnp.bfloat16)
    w2b = W2.astype(jnp.bfloat16)
    y, loss = pl.pallas_call(
        _moe_block,
        grid=(GRID,),
        in_specs=[
            pl.BlockSpec((TN, D), lambda t: (t, 0)),
            pl.BlockSpec((D, E), lambda t: (0, 0)),
            pl.BlockSpec((E, D, H), lambda t: (0, 0, 0)),
            pl.BlockSpec((E, H), lambda t: (0, 0)),
            pl.BlockSpec((E, H, D), lambda t: (0, 0, 0)),
            pl.BlockSpec((E, D), lambda t: (0, 0)),
            pl.BlockSpec((1, D), lambda t: (0, 0)),
        ],
        out_specs=[
            pl.BlockSpec((TN, D), lambda t: (t, 0)),
            pl.BlockSpec((1, 1), lambda t: (0, 0)),
        ],
        out_shape=[
            jax.ShapeDtypeStruct((P, D), jnp.float32),
            jax.ShapeDtypeStruct((1, 1), jnp.float32),
        ],
        scratch_shapes=[pltpu.VMEM((2, E), jnp.float32)],
    )(x_flat, w_gate, w1b, b1, w2b, b2, gamma.reshape(1, D))
    return y.reshape(B, P, D), loss[0, 0]
